# baseline (speedup 1.0000x reference)
import functools

import jax
import jax.numpy as jnp
from jax import lax
from jax.experimental import pallas as pl
from jax.experimental.pallas import tpu as pltpu
from jax.experimental.pallas import tpu_sc as plsc

_NC = 2
_NS = 16
_K = 80
_DW = 8


def _pad_rows(n):
  q = _NS * 8
  return ((n + q - 1) // q) * q


def _make_sc_agg(n, e, d, with_deg):
  nw = _NC * _NS
  chunks = e // (nw * _K)
  assert chunks * nw * _K == e
  np_ = _pad_rows(n)
  rows_pt = np_ // _NS

  mesh = plsc.VectorSubcoreMesh(core_axis_name="c", subcore_axis_name="s")

  out_type = [jax.ShapeDtypeStruct((_NC, np_, d), jnp.float32)]
  scratch = [
      pltpu.VMEM((chunks, _K), jnp.int32),
      pltpu.VMEM((chunks, _K), jnp.int32),
      pltpu.VMEM((_K, d), jnp.float32),
      pltpu.VMEM((_K, d), jnp.float32),
      pltpu.VMEM_SHARED((np_, d), jnp.float32),
      pltpu.SemaphoreType.DMA,
      pltpu.SemaphoreType.DMA,
      pltpu.SemaphoreType.DMA,
      pltpu.SemaphoreType.DMA,
  ]
  if with_deg:
    out_type.append(jax.ShapeDtypeStruct((_NC, np_, _DW), jnp.float32))
    scratch += [
        pltpu.VMEM((_K, _DW), jnp.float32),
        pltpu.VMEM_SHARED((np_, _DW), jnp.float32),
    ]

  @functools.partial(
      pl.kernel, mesh=mesh, out_type=out_type, scratch_types=scratch,
      compiler_params=pltpu.CompilerParams(use_tc_tiling_on_sc=False))
  def body(table, edges4, zacc, zdeg, ones, *refs):
    if with_deg:
      (agg_out, deg_out, srcs_v, dsts_v, rows0_v, rows1_v, acc_sh,
       gsem0, gsem1, ssem0, ssem1, ones_v, deg_sh) = refs
    else:
      (agg_out, srcs_v, dsts_v, rows0_v, rows1_v, acc_sh,
       gsem0, gsem1, ssem0, ssem1) = refs
    bufs = ((rows0_v, gsem0, ssem0), (rows1_v, gsem1, ssem1))
    cid = lax.axis_index("c")
    sid = lax.axis_index("s")
    wid = cid * _NS + sid

    r0 = sid * rows_pt
    pltpu.sync_copy(edges4.at[0, wid], srcs_v)
    pltpu.sync_copy(edges4.at[1, wid], dsts_v)
    pltpu.sync_copy(zacc, acc_sh.at[pl.ds(r0, rows_pt)])
    if with_deg:
      pltpu.sync_copy(zdeg, deg_sh.at[pl.ds(r0, rows_pt)])
      pltpu.sync_copy(ones, ones_v)
    plsc.subcore_barrier()

    def gather(j, b):
      rows_v, gsem, _ = bufs[b]
      pltpu.make_async_copy(table.at[srcs_v.at[j]], rows_v, gsem).start()

    def scatter_start(j, b):
      rows_v, _, ssem = bufs[b]
      pltpu.async_copy(rows_v, acc_sh.at[dsts_v.at[j]], ssem, add=True)
      if with_deg:
        pltpu.async_copy(ones_v, deg_sh.at[dsts_v.at[j]], ssem, add=True)

    def scatter_descs(j, b):
      rows_v, _, ssem = bufs[b]
      ds = [pltpu.make_async_copy(rows_v, acc_sh.at[dsts_v.at[j]], ssem)]
      if with_deg:
        ds.append(pltpu.make_async_copy(ones_v, deg_sh.at[dsts_v.at[j]], ssem))
      return ds

    def step(j, b, last):
      rows_v, gsem, _ = bufs[b]
      pltpu.make_async_copy(table.at[srcs_v.at[j]], rows_v, gsem).wait()
      scatter_start(j, b)
      if not last:
        ob = 1 - b

        @pl.when(j >= 1)
        def _():
          for dsc in scatter_descs(j - 1, ob):
            dsc.wait()

        @pl.when(j + 1 < chunks)
        def _():
          gather(j + 1, ob)

    gather(0, 0)

    def pair(i, carry):
      step(2 * i, 0, False)
      step(2 * i + 1, 1, False)
      return carry

    lax.fori_loop(0, chunks // 2, pair, 0)
    if chunks % 2:
      step(chunks - 1, 0, True)
      for dsc in scatter_descs(chunks - 2, 1):
        dsc.wait()
    for dsc in scatter_descs(chunks - 1, (chunks - 1) % 2):
      dsc.wait()
    plsc.subcore_barrier()

    pltpu.sync_copy(acc_sh.at[pl.ds(r0, rows_pt)],
                    agg_out.at[cid, pl.ds(r0, rows_pt)])
    if with_deg:
      pltpu.sync_copy(deg_sh.at[pl.ds(r0, rows_pt)],
                      deg_out.at[cid, pl.ds(r0, rows_pt)])

  return body


def _tc_layer1(x, agg, deg, w_self1, w_neigh1, b1, w_self2, w_neigh2):
  n, d = x.shape
  h = w_self1.shape[0]
  c = w_self2.shape[0]
  bn = 1000
  grid = (n // bn,)

  def tcb(x_b, agg_b, deg_b, ws1, wn1, b1_b, ws2, wn2,
          h1_b, h1r_b, p2_b, hs2_b):
    degs = jnp.maximum(deg_b[0, :, 0] + deg_b[1, :, 0], 1.0)
    mean = (agg_b[0] + agg_b[1]) / degs[:, None]
    dn = (((1,), (1,)), ((), ()))
    h1 = (lax.dot_general(x_b[...], ws1[...], dn,
                          preferred_element_type=jnp.float32)
          + lax.dot_general(mean, wn1[...], dn,
                            preferred_element_type=jnp.float32)
          + b1_b[...])
    h1_b[...] = h1
    h1r = jnp.maximum(h1, 0.0)
    h1r_b[...] = h1r
    p2_b[...] = lax.dot_general(h1r, wn2[...], dn,
                                preferred_element_type=jnp.float32)
    hs2_b[...] = lax.dot_general(h1r, ws2[...], dn,
                                 preferred_element_type=jnp.float32)

  return pl.pallas_call(
      tcb,
      grid=grid,
      in_specs=[
          pl.BlockSpec((bn, d), lambda i: (i, 0)),
          pl.BlockSpec((_NC, bn, d), lambda i: (0, i, 0)),
          pl.BlockSpec((_NC, bn, _DW), lambda i: (0, i, 0)),
          pl.BlockSpec((h, d), lambda i: (0, 0)),
          pl.BlockSpec((h, d), lambda i: (0, 0)),
          pl.BlockSpec((1, h), lambda i: (0, 0)),
          pl.BlockSpec((c, h), lambda i: (0, 0)),
          pl.BlockSpec((c, h), lambda i: (0, 0)),
      ],
      out_specs=[
          pl.BlockSpec((bn, h), lambda i: (i, 0)),
          pl.BlockSpec((bn, h), lambda i: (i, 0)),
          pl.BlockSpec((bn, c), lambda i: (i, 0)),
          pl.BlockSpec((bn, c), lambda i: (i, 0)),
      ],
      out_shape=[
          jax.ShapeDtypeStruct((n, h), jnp.float32),
          jax.ShapeDtypeStruct((n, h), jnp.float32),
          jax.ShapeDtypeStruct((n, c), jnp.float32),
          jax.ShapeDtypeStruct((n, c), jnp.float32),
      ],
  )(x, agg, deg, w_self1, w_neigh1, b1.reshape(1, h), w_self2, w_neigh2)


def _tc_layer2(hs2, agg2, deg, b2):
  n, c = hs2.shape

  def tcc(hs2_b, agg2_b, deg_b, b2_b, h2_b):
    degs = jnp.maximum(deg_b[0, :, 0] + deg_b[1, :, 0], 1.0)
    h2_b[...] = hs2_b[...] + (agg2_b[0] + agg2_b[1]) / degs[:, None] + b2_b[...]

  return pl.pallas_call(
      tcc,
      grid=(1,),
      in_specs=[
          pl.BlockSpec((n, c), lambda i: (0, 0)),
          pl.BlockSpec((_NC, n, c), lambda i: (0, 0, 0)),
          pl.BlockSpec((_NC, n, _DW), lambda i: (0, 0, 0)),
          pl.BlockSpec((1, c), lambda i: (0, 0)),
      ],
      out_specs=pl.BlockSpec((n, c), lambda i: (0, 0)),
      out_shape=jax.ShapeDtypeStruct((n, c), jnp.float32),
  )(hs2, agg2, deg, b2.reshape(1, c))


def kernel(x, edge_index, W_self1, W_neigh1, b1, W_self2, W_neigh2, b2):
  n, d = x.shape
  e = edge_index.shape[1]
  c = W_self2.shape[0]
  rows_pt = _pad_rows(n) // _NS

  e4 = edge_index.reshape(2, _NC * _NS, e // (_NC * _NS * _K), _K)
  zacc = jnp.zeros((rows_pt, d), jnp.float32)
  zdeg = jnp.zeros((rows_pt, _DW), jnp.float32)
  zacc2 = jnp.zeros((rows_pt, c), jnp.float32)
  ones = jnp.ones((_K, _DW), jnp.float32)

  agg1, deg = _make_sc_agg(n, e, d, True)(x, e4, zacc, zdeg, ones)
  h1, h1r, p2, hs2 = _tc_layer1(x, agg1, deg, W_self1, W_neigh1, b1,
                                W_self2, W_neigh2)
  agg2 = _make_sc_agg(n, e, c, False)(p2, e4, zacc2, zdeg, ones)[0]
  h2 = _tc_layer2(hs2, agg2, deg, b2)
  return (h2, h1, h1r)


# speedup vs baseline: 1.2666x; 1.2666x over previous
import functools

import jax
import jax.numpy as jnp
from jax import lax
from jax.experimental import pallas as pl
from jax.experimental.pallas import tpu as pltpu
from jax.experimental.pallas import tpu_sc as plsc

_NC = 2
_NS = 16
_K = 80
_DW = 8


def _pad_rows(n):
  q = _NS * 8
  return ((n + q - 1) // q) * q


def _make_sc_agg(n, e, d, with_deg):
  nw = _NC * _NS
  chunks = e // (nw * _K)
  assert chunks * nw * _K == e
  np_ = _pad_rows(n)
  rows_pt = np_ // _NS

  mesh = plsc.VectorSubcoreMesh(core_axis_name="c", subcore_axis_name="s")

  out_type = [jax.ShapeDtypeStruct((_NC, np_, d), jnp.float32)]
  scratch = [
      pltpu.VMEM((chunks, _K), jnp.int32),
      pltpu.VMEM((chunks, _K), jnp.int32),
      pltpu.VMEM((_K, d), jnp.float32),
      pltpu.VMEM((_K, d), jnp.float32),
      pltpu.VMEM_SHARED((np_, d), jnp.float32),
      pltpu.SemaphoreType.DMA,
      pltpu.SemaphoreType.DMA,
  ]
  if with_deg:
    out_type.append(jax.ShapeDtypeStruct((_NC, np_, _DW), jnp.float32))
    scratch += [
        pltpu.VMEM((_K, _DW), jnp.float32),
        pltpu.VMEM_SHARED((np_, _DW), jnp.float32),
    ]

  @functools.partial(
      pl.kernel, mesh=mesh, out_type=out_type, scratch_types=scratch,
      compiler_params=pltpu.CompilerParams(use_tc_tiling_on_sc=False))
  def body(table, edges4, zacc, zdeg, ones, *refs):
    if with_deg:
      (agg_out, deg_out, srcs_v, dsts_v, rows0_v, rows1_v, acc_sh,
       gsem0, gsem1, ones_v, deg_sh) = refs
    else:
      (agg_out, srcs_v, dsts_v, rows0_v, rows1_v, acc_sh, gsem0, gsem1) = refs
    bufs = ((rows0_v, gsem0, None), (rows1_v, gsem1, None))
    cid = lax.axis_index("c")
    sid = lax.axis_index("s")
    wid = cid * _NS + sid

    r0 = sid * rows_pt
    pltpu.sync_copy(edges4.at[0, wid], srcs_v)
    pltpu.sync_copy(edges4.at[1, wid], dsts_v)
    pltpu.sync_copy(zacc, acc_sh.at[pl.ds(r0, rows_pt)])
    if with_deg:
      pltpu.sync_copy(zdeg, deg_sh.at[pl.ds(r0, rows_pt)])
      pltpu.sync_copy(ones, ones_v)
    plsc.subcore_barrier()

    def gather(j, b):
      rows_v, gsem, _ = bufs[b]
      pltpu.make_async_copy(table.at[srcs_v.at[j]], rows_v, gsem).start()

    def wait_scatter(j, b):
      rows_v, gsem, _ = bufs[b]
      pltpu.make_async_copy(table.at[srcs_v.at[j]], rows_v, gsem).wait()
      pltpu.sync_copy(rows_v, acc_sh.at[dsts_v.at[j]], add=True)
      if with_deg:
        pltpu.sync_copy(ones_v, deg_sh.at[dsts_v.at[j]], add=True)

    gather(0, 0)

    def pair(i, carry):
      j0 = 2 * i
      gather(j0 + 1, 1)
      wait_scatter(j0, 0)

      @pl.when(j0 + 2 < chunks)
      def _():
        gather(j0 + 2, 0)

      wait_scatter(j0 + 1, 1)
      return carry

    lax.fori_loop(0, chunks // 2, pair, 0)
    if chunks % 2:
      wait_scatter(chunks - 1, 0)
    plsc.subcore_barrier()

    pltpu.sync_copy(acc_sh.at[pl.ds(r0, rows_pt)],
                    agg_out.at[cid, pl.ds(r0, rows_pt)])
    if with_deg:
      pltpu.sync_copy(deg_sh.at[pl.ds(r0, rows_pt)],
                      deg_out.at[cid, pl.ds(r0, rows_pt)])

  return body


def _tc_layer1(x, agg, deg, w_self1, w_neigh1, b1, w_self2, w_neigh2):
  n, d = x.shape
  h = w_self1.shape[0]
  c = w_self2.shape[0]
  bn = 1000
  grid = (n // bn,)

  def tcb(x_b, agg_b, deg_b, ws1, wn1, b1_b, ws2, wn2,
          h1_b, h1r_b, p2_b, hs2_b):
    degs = jnp.maximum(deg_b[0, :, 0] + deg_b[1, :, 0], 1.0)
    mean = (agg_b[0] + agg_b[1]) / degs[:, None]
    dn = (((1,), (1,)), ((), ()))
    h1 = (lax.dot_general(x_b[...], ws1[...], dn,
                          preferred_element_type=jnp.float32)
          + lax.dot_general(mean, wn1[...], dn,
                            preferred_element_type=jnp.float32)
          + b1_b[...])
    h1_b[...] = h1
    h1r = jnp.maximum(h1, 0.0)
    h1r_b[...] = h1r
    p2_b[...] = lax.dot_general(h1r, wn2[...], dn,
                                preferred_element_type=jnp.float32)
    hs2_b[...] = lax.dot_general(h1r, ws2[...], dn,
                                 preferred_element_type=jnp.float32)

  return pl.pallas_call(
      tcb,
      grid=grid,
      in_specs=[
          pl.BlockSpec((bn, d), lambda i: (i, 0)),
          pl.BlockSpec((_NC, bn, d), lambda i: (0, i, 0)),
          pl.BlockSpec((_NC, bn, _DW), lambda i: (0, i, 0)),
          pl.BlockSpec((h, d), lambda i: (0, 0)),
          pl.BlockSpec((h, d), lambda i: (0, 0)),
          pl.BlockSpec((1, h), lambda i: (0, 0)),
          pl.BlockSpec((c, h), lambda i: (0, 0)),
          pl.BlockSpec((c, h), lambda i: (0, 0)),
      ],
      out_specs=[
          pl.BlockSpec((bn, h), lambda i: (i, 0)),
          pl.BlockSpec((bn, h), lambda i: (i, 0)),
          pl.BlockSpec((bn, c), lambda i: (i, 0)),
          pl.BlockSpec((bn, c), lambda i: (i, 0)),
      ],
      out_shape=[
          jax.ShapeDtypeStruct((n, h), jnp.float32),
          jax.ShapeDtypeStruct((n, h), jnp.float32),
          jax.ShapeDtypeStruct((n, c), jnp.float32),
          jax.ShapeDtypeStruct((n, c), jnp.float32),
      ],
  )(x, agg, deg, w_self1, w_neigh1, b1.reshape(1, h), w_self2, w_neigh2)


def _tc_layer2(hs2, agg2, deg, b2):
  n, c = hs2.shape

  def tcc(hs2_b, agg2_b, deg_b, b2_b, h2_b):
    degs = jnp.maximum(deg_b[0, :, 0] + deg_b[1, :, 0], 1.0)
    h2_b[...] = hs2_b[...] + (agg2_b[0] + agg2_b[1]) / degs[:, None] + b2_b[...]

  return pl.pallas_call(
      tcc,
      grid=(1,),
      in_specs=[
          pl.BlockSpec((n, c), lambda i: (0, 0)),
          pl.BlockSpec((_NC, n, c), lambda i: (0, 0, 0)),
          pl.BlockSpec((_NC, n, _DW), lambda i: (0, 0, 0)),
          pl.BlockSpec((1, c), lambda i: (0, 0)),
      ],
      out_specs=pl.BlockSpec((n, c), lambda i: (0, 0)),
      out_shape=jax.ShapeDtypeStruct((n, c), jnp.float32),
  )(hs2, agg2, deg, b2.reshape(1, c))


def kernel(x, edge_index, W_self1, W_neigh1, b1, W_self2, W_neigh2, b2):
  n, d = x.shape
  e = edge_index.shape[1]
  c = W_self2.shape[0]
  rows_pt = _pad_rows(n) // _NS

  e4 = edge_index.reshape(2, _NC * _NS, e // (_NC * _NS * _K), _K)
  zacc = jnp.zeros((rows_pt, d), jnp.float32)
  zdeg = jnp.zeros((rows_pt, _DW), jnp.float32)
  zacc2 = jnp.zeros((rows_pt, c), jnp.float32)
  ones = jnp.ones((_K, _DW), jnp.float32)

  agg1, deg = _make_sc_agg(n, e, d, True)(x, e4, zacc, zdeg, ones)
  h1, h1r, p2, hs2 = _tc_layer1(x, agg1, deg, W_self1, W_neigh1, b1,
                                W_self2, W_neigh2)
  agg2 = _make_sc_agg(n, e, c, False)(p2, e4, zacc2, zdeg, ones)[0]
  h2 = _tc_layer2(hs2, agg2, deg, b2)
  return (h2, h1, h1r)


# speedup vs baseline: 1.2808x; 1.0112x over previous
import functools

import jax
import jax.numpy as jnp
from jax import lax
from jax.experimental import pallas as pl
from jax.experimental.pallas import tpu as pltpu
from jax.experimental.pallas import tpu_sc as plsc

_NC = 2
_NS = 16
_K = 80
_DW = 8


def _pad_rows(n):
  q = _NS * 8
  return ((n + q - 1) // q) * q


def _make_sc_agg(n, e, d, with_deg):
  nw = _NC * _NS
  chunks = e // (nw * _K)
  assert chunks * nw * _K == e
  np_ = _pad_rows(n)
  rows_pt = np_ // _NS

  mesh = plsc.VectorSubcoreMesh(core_axis_name="c", subcore_axis_name="s")

  out_type = [jax.ShapeDtypeStruct((_NC, np_, d), jnp.float32)]
  scratch = [
      pltpu.VMEM((chunks, _K), jnp.int32),
      pltpu.VMEM((chunks, _K), jnp.int32),
      pltpu.VMEM((_K, d), jnp.float32),
      pltpu.VMEM((_K, d), jnp.float32),
      pltpu.VMEM_SHARED((np_, d), jnp.float32),
      pltpu.SemaphoreType.DMA,
      pltpu.SemaphoreType.DMA,
  ]
  if with_deg:
    out_type.append(jax.ShapeDtypeStruct((_NC, np_, _DW), jnp.float32))
    scratch += [
        pltpu.VMEM((_K, _DW), jnp.float32),
        pltpu.VMEM_SHARED((np_, _DW), jnp.float32),
        pltpu.SemaphoreType.DMA,
    ]

  @functools.partial(
      pl.kernel, mesh=mesh, out_type=out_type, scratch_types=scratch,
      compiler_params=pltpu.CompilerParams(use_tc_tiling_on_sc=False))
  def body(table, edges4, zacc, zdeg, ones, *refs):
    if with_deg:
      (agg_out, deg_out, srcs_v, dsts_v, rows0_v, rows1_v, acc_sh,
       gsem0, gsem1, ones_v, deg_sh, osem) = refs
    else:
      (agg_out, srcs_v, dsts_v, rows0_v, rows1_v, acc_sh, gsem0, gsem1) = refs
    bufs = ((rows0_v, gsem0, None), (rows1_v, gsem1, None))
    cid = lax.axis_index("c")
    sid = lax.axis_index("s")
    wid = cid * _NS + sid

    r0 = sid * rows_pt
    pltpu.sync_copy(edges4.at[0, wid], srcs_v)
    pltpu.sync_copy(edges4.at[1, wid], dsts_v)
    pltpu.sync_copy(zacc, acc_sh.at[pl.ds(r0, rows_pt)])
    if with_deg:
      pltpu.sync_copy(zdeg, deg_sh.at[pl.ds(r0, rows_pt)])
      pltpu.sync_copy(ones, ones_v)
    plsc.subcore_barrier()

    def gather(j, b):
      rows_v, gsem, _ = bufs[b]
      pltpu.make_async_copy(table.at[srcs_v.at[j]], rows_v, gsem).start()

    def wait_scatter(j, b):
      rows_v, gsem, _ = bufs[b]
      pltpu.make_async_copy(table.at[srcs_v.at[j]], rows_v, gsem).wait()
      pltpu.sync_copy(rows_v, acc_sh.at[dsts_v.at[j]], add=True)
      if with_deg:
        pltpu.async_copy(ones_v, deg_sh.at[dsts_v.at[j]], osem, add=True)

    gather(0, 0)

    def pair(i, carry):
      j0 = 2 * i
      gather(j0 + 1, 1)
      wait_scatter(j0, 0)

      @pl.when(j0 + 2 < chunks)
      def _():
        gather(j0 + 2, 0)

      wait_scatter(j0 + 1, 1)
      return carry

    lax.fori_loop(0, chunks // 2, pair, 0)
    if chunks % 2:
      wait_scatter(chunks - 1, 0)
    if with_deg:
      def drain(j, carry):
        pltpu.make_async_copy(ones_v, deg_sh.at[dsts_v.at[j]], osem).wait()
        return carry
      lax.fori_loop(0, chunks, drain, 0)
    plsc.subcore_barrier()

    pltpu.sync_copy(acc_sh.at[pl.ds(r0, rows_pt)],
                    agg_out.at[cid, pl.ds(r0, rows_pt)])
    if with_deg:
      pltpu.sync_copy(deg_sh.at[pl.ds(r0, rows_pt)],
                      deg_out.at[cid, pl.ds(r0, rows_pt)])

  return body


def _tc_layer1(x, agg, deg, w_self1, w_neigh1, b1, w_self2, w_neigh2):
  n, d = x.shape
  h = w_self1.shape[0]
  c = w_self2.shape[0]
  bn = 1000
  grid = (n // bn,)

  def tcb(x_b, agg_b, deg_b, ws1, wn1, b1_b, ws2, wn2,
          h1_b, h1r_b, p2_b, hs2_b):
    degs = jnp.maximum(deg_b[0, :, 0] + deg_b[1, :, 0], 1.0)
    mean = (agg_b[0] + agg_b[1]) / degs[:, None]
    dn = (((1,), (1,)), ((), ()))
    h1 = (lax.dot_general(x_b[...], ws1[...], dn,
                          preferred_element_type=jnp.float32)
          + lax.dot_general(mean, wn1[...], dn,
                            preferred_element_type=jnp.float32)
          + b1_b[...])
    h1_b[...] = h1
    h1r = jnp.maximum(h1, 0.0)
    h1r_b[...] = h1r
    p2_b[...] = lax.dot_general(h1r, wn2[...], dn,
                                preferred_element_type=jnp.float32)
    hs2_b[...] = lax.dot_general(h1r, ws2[...], dn,
                                 preferred_element_type=jnp.float32)

  return pl.pallas_call(
      tcb,
      grid=grid,
      in_specs=[
          pl.BlockSpec((bn, d), lambda i: (i, 0)),
          pl.BlockSpec((_NC, bn, d), lambda i: (0, i, 0)),
          pl.BlockSpec((_NC, bn, _DW), lambda i: (0, i, 0)),
          pl.BlockSpec((h, d), lambda i: (0, 0)),
          pl.BlockSpec((h, d), lambda i: (0, 0)),
          pl.BlockSpec((1, h), lambda i: (0, 0)),
          pl.BlockSpec((c, h), lambda i: (0, 0)),
          pl.BlockSpec((c, h), lambda i: (0, 0)),
      ],
      out_specs=[
          pl.BlockSpec((bn, h), lambda i: (i, 0)),
          pl.BlockSpec((bn, h), lambda i: (i, 0)),
          pl.BlockSpec((bn, c), lambda i: (i, 0)),
          pl.BlockSpec((bn, c), lambda i: (i, 0)),
      ],
      out_shape=[
          jax.ShapeDtypeStruct((n, h), jnp.float32),
          jax.ShapeDtypeStruct((n, h), jnp.float32),
          jax.ShapeDtypeStruct((n, c), jnp.float32),
          jax.ShapeDtypeStruct((n, c), jnp.float32),
      ],
  )(x, agg, deg, w_self1, w_neigh1, b1.reshape(1, h), w_self2, w_neigh2)


def _tc_layer2(hs2, agg2, deg, b2):
  n, c = hs2.shape

  def tcc(hs2_b, agg2_b, deg_b, b2_b, h2_b):
    degs = jnp.maximum(deg_b[0, :, 0] + deg_b[1, :, 0], 1.0)
    h2_b[...] = hs2_b[...] + (agg2_b[0] + agg2_b[1]) / degs[:, None] + b2_b[...]

  return pl.pallas_call(
      tcc,
      grid=(1,),
      in_specs=[
          pl.BlockSpec((n, c), lambda i: (0, 0)),
          pl.BlockSpec((_NC, n, c), lambda i: (0, 0, 0)),
          pl.BlockSpec((_NC, n, _DW), lambda i: (0, 0, 0)),
          pl.BlockSpec((1, c), lambda i: (0, 0)),
      ],
      out_specs=pl.BlockSpec((n, c), lambda i: (0, 0)),
      out_shape=jax.ShapeDtypeStruct((n, c), jnp.float32),
  )(hs2, agg2, deg, b2.reshape(1, c))


def kernel(x, edge_index, W_self1, W_neigh1, b1, W_self2, W_neigh2, b2):
  n, d = x.shape
  e = edge_index.shape[1]
  c = W_self2.shape[0]
  rows_pt = _pad_rows(n) // _NS

  e4 = edge_index.reshape(2, _NC * _NS, e // (_NC * _NS * _K), _K)
  zacc = jnp.zeros((rows_pt, d), jnp.float32)
  zdeg = jnp.zeros((rows_pt, _DW), jnp.float32)
  zacc2 = jnp.zeros((rows_pt, c), jnp.float32)
  ones = jnp.ones((_K, _DW), jnp.float32)

  agg1, deg = _make_sc_agg(n, e, d, True)(x, e4, zacc, zdeg, ones)
  h1, h1r, p2, hs2 = _tc_layer1(x, agg1, deg, W_self1, W_neigh1, b1,
                                W_self2, W_neigh2)
  agg2 = _make_sc_agg(n, e, c, False)(p2, e4, zacc2, zdeg, ones)[0]
  h2 = _tc_layer2(hs2, agg2, deg, b2)
  return (h2, h1, h1r)


# speedup vs baseline: 1.3761x; 1.0744x over previous
import functools

import jax
import jax.numpy as jnp
from jax import lax
from jax.experimental import pallas as pl
from jax.experimental.pallas import tpu as pltpu
from jax.experimental.pallas import tpu_sc as plsc

_NC = 2
_NS = 16
_K = 80
_DW = 8
_G2 = 5


def _pad_rows(n):
  q = _NS * 8
  return ((n + q - 1) // q) * q


def _make_sc_agg(n, e, d, with_deg, group=1):
  nw = _NC * _NS
  chunks = e // (nw * _K)
  assert chunks * nw * _K == e
  steps = chunks // group
  assert steps * group == chunks
  np_ = _pad_rows(n)
  rows_pt = np_ // _NS

  mesh = plsc.VectorSubcoreMesh(core_axis_name="c", subcore_axis_name="s")

  out_type = [jax.ShapeDtypeStruct((_NC, np_, d), jnp.float32)]
  scratch = [
      pltpu.VMEM((steps, group * _K), jnp.int32),
      pltpu.VMEM((chunks, _K), jnp.int32),
      pltpu.VMEM((group * _K, d), jnp.float32),
      pltpu.VMEM((group * _K, d), jnp.float32),
      pltpu.VMEM_SHARED((np_, d), jnp.float32),
      pltpu.SemaphoreType.DMA,
      pltpu.SemaphoreType.DMA,
  ]
  if with_deg:
    out_type.append(jax.ShapeDtypeStruct((_NC, np_, _DW), jnp.float32))
    scratch += [
        pltpu.VMEM((_K, _DW), jnp.float32),
        pltpu.VMEM_SHARED((np_, _DW), jnp.float32),
        pltpu.SemaphoreType.DMA,
    ]

  @functools.partial(
      pl.kernel, mesh=mesh, out_type=out_type, scratch_types=scratch,
      compiler_params=pltpu.CompilerParams(use_tc_tiling_on_sc=False))
  def body(table, esrc, edst, zacc, zdeg, ones, *refs):
    if with_deg:
      (agg_out, deg_out, srcs_v, dsts_v, rows0_v, rows1_v, acc_sh,
       gsem0, gsem1, ones_v, deg_sh, osem) = refs
    else:
      (agg_out, srcs_v, dsts_v, rows0_v, rows1_v, acc_sh, gsem0, gsem1) = refs
    bufs = ((rows0_v, gsem0, None), (rows1_v, gsem1, None))
    cid = lax.axis_index("c")
    sid = lax.axis_index("s")
    wid = cid * _NS + sid

    r0 = sid * rows_pt
    pltpu.sync_copy(esrc.at[wid], srcs_v)
    pltpu.sync_copy(edst.at[wid], dsts_v)
    pltpu.sync_copy(zacc, acc_sh.at[pl.ds(r0, rows_pt)])
    if with_deg:
      pltpu.sync_copy(zdeg, deg_sh.at[pl.ds(r0, rows_pt)])
      pltpu.sync_copy(ones, ones_v)
    plsc.subcore_barrier()

    def gather(j, b):
      rows_v, gsem, _ = bufs[b]
      pltpu.make_async_copy(table.at[srcs_v.at[j]], rows_v, gsem).start()

    def wait_scatter(j, b):
      rows_v, gsem, _ = bufs[b]
      pltpu.make_async_copy(table.at[srcs_v.at[j]], rows_v, gsem).wait()
      for k in range(group):
        rv = rows_v if group == 1 else rows_v.at[pl.ds(_K * k, _K)]
        pltpu.sync_copy(rv, acc_sh.at[dsts_v.at[group * j + k]], add=True)
        if with_deg:
          pltpu.async_copy(ones_v, deg_sh.at[dsts_v.at[group * j + k]], osem,
                           add=True)

    gather(0, 0)

    def pair(i, carry):
      j0 = 2 * i
      gather(j0 + 1, 1)
      wait_scatter(j0, 0)

      @pl.when(j0 + 2 < steps)
      def _():
        gather(j0 + 2, 0)

      wait_scatter(j0 + 1, 1)
      return carry

    lax.fori_loop(0, steps // 2, pair, 0)
    if steps % 2:
      wait_scatter(steps - 1, 0)
    if with_deg:
      def drain(j, carry):
        pltpu.make_async_copy(ones_v, deg_sh.at[dsts_v.at[j]], osem).wait()
        return carry
      lax.fori_loop(0, chunks, drain, 0)
    plsc.subcore_barrier()

    pltpu.sync_copy(acc_sh.at[pl.ds(r0, rows_pt)],
                    agg_out.at[cid, pl.ds(r0, rows_pt)])
    if with_deg:
      pltpu.sync_copy(deg_sh.at[pl.ds(r0, rows_pt)],
                      deg_out.at[cid, pl.ds(r0, rows_pt)])

  return body


def _tc_layer1(x, agg, deg, w_self1, w_neigh1, b1, w_self2, w_neigh2):
  n, d = x.shape
  h = w_self1.shape[0]
  c = w_self2.shape[0]
  bn = 1000
  grid = (n // bn,)

  def tcb(x_b, agg_b, deg_b, ws1, wn1, b1_b, ws2, wn2,
          h1_b, h1r_b, p2_b, hs2_b):
    degs = jnp.maximum(deg_b[0, :, 0] + deg_b[1, :, 0], 1.0)
    mean = (agg_b[0] + agg_b[1]) / degs[:, None]
    dn = (((1,), (1,)), ((), ()))
    h1 = (lax.dot_general(x_b[...], ws1[...], dn,
                          preferred_element_type=jnp.float32)
          + lax.dot_general(mean, wn1[...], dn,
                            preferred_element_type=jnp.float32)
          + b1_b[...])
    h1_b[...] = h1
    h1r = jnp.maximum(h1, 0.0)
    h1r_b[...] = h1r
    p2_b[...] = lax.dot_general(h1r, wn2[...], dn,
                                preferred_element_type=jnp.float32)
    hs2_b[...] = lax.dot_general(h1r, ws2[...], dn,
                                 preferred_element_type=jnp.float32)

  return pl.pallas_call(
      tcb,
      grid=grid,
      in_specs=[
          pl.BlockSpec((bn, d), lambda i: (i, 0)),
          pl.BlockSpec((_NC, bn, d), lambda i: (0, i, 0)),
          pl.BlockSpec((_NC, bn, _DW), lambda i: (0, i, 0)),
          pl.BlockSpec((h, d), lambda i: (0, 0)),
          pl.BlockSpec((h, d), lambda i: (0, 0)),
          pl.BlockSpec((1, h), lambda i: (0, 0)),
          pl.BlockSpec((c, h), lambda i: (0, 0)),
          pl.BlockSpec((c, h), lambda i: (0, 0)),
      ],
      out_specs=[
          pl.BlockSpec((bn, h), lambda i: (i, 0)),
          pl.BlockSpec((bn, h), lambda i: (i, 0)),
          pl.BlockSpec((bn, c), lambda i: (i, 0)),
          pl.BlockSpec((bn, c), lambda i: (i, 0)),
      ],
      out_shape=[
          jax.ShapeDtypeStruct((n, h), jnp.float32),
          jax.ShapeDtypeStruct((n, h), jnp.float32),
          jax.ShapeDtypeStruct((n, c), jnp.float32),
          jax.ShapeDtypeStruct((n, c), jnp.float32),
      ],
  )(x, agg, deg, w_self1, w_neigh1, b1.reshape(1, h), w_self2, w_neigh2)


def _tc_layer2(hs2, agg2, deg, b2):
  n, c = hs2.shape

  def tcc(hs2_b, agg2_b, deg_b, b2_b, h2_b):
    degs = jnp.maximum(deg_b[0, :, 0] + deg_b[1, :, 0], 1.0)
    h2_b[...] = hs2_b[...] + (agg2_b[0] + agg2_b[1]) / degs[:, None] + b2_b[...]

  return pl.pallas_call(
      tcc,
      grid=(1,),
      in_specs=[
          pl.BlockSpec((n, c), lambda i: (0, 0)),
          pl.BlockSpec((_NC, n, c), lambda i: (0, 0, 0)),
          pl.BlockSpec((_NC, n, _DW), lambda i: (0, 0, 0)),
          pl.BlockSpec((1, c), lambda i: (0, 0)),
      ],
      out_specs=pl.BlockSpec((n, c), lambda i: (0, 0)),
      out_shape=jax.ShapeDtypeStruct((n, c), jnp.float32),
  )(hs2, agg2, deg, b2.reshape(1, c))


def kernel(x, edge_index, W_self1, W_neigh1, b1, W_self2, W_neigh2, b2):
  n, d = x.shape
  e = edge_index.shape[1]
  c = W_self2.shape[0]
  rows_pt = _pad_rows(n) // _NS

  nw = _NC * _NS
  chunks = e // (nw * _K)
  esrc1 = edge_index[0].reshape(nw, chunks, _K)
  esrc2 = edge_index[0].reshape(nw, chunks // _G2, _G2 * _K)
  edst = edge_index[1].reshape(nw, chunks, _K)
  zacc = jnp.zeros((rows_pt, d), jnp.float32)
  zdeg = jnp.zeros((rows_pt, _DW), jnp.float32)
  zacc2 = jnp.zeros((rows_pt, c), jnp.float32)
  ones = jnp.ones((_K, _DW), jnp.float32)

  agg1, deg = _make_sc_agg(n, e, d, True)(x, esrc1, edst, zacc, zdeg, ones)
  h1, h1r, p2, hs2 = _tc_layer1(x, agg1, deg, W_self1, W_neigh1, b1,
                                W_self2, W_neigh2)
  agg2 = _make_sc_agg(n, e, c, False, group=_G2)(
      p2, esrc2, edst, zacc2, zdeg, ones)[0]
  h2 = _tc_layer2(hs2, agg2, deg, b2)
  return (h2, h1, h1r)


# speedup vs baseline: 1.3992x; 1.0168x over previous
import functools

import jax
import jax.numpy as jnp
from jax import lax
from jax.experimental import pallas as pl
from jax.experimental.pallas import tpu as pltpu
from jax.experimental.pallas import tpu_sc as plsc

_NC = 2
_NS = 16
_K = 80
_DW = 8
_G2 = 5


def _pad_rows(n):
  q = _NS * 8
  return ((n + q - 1) // q) * q


def _make_sc_agg(n, e, d, with_deg, group=1):
  nw = _NC * _NS
  chunks = e // (nw * _K)
  assert chunks * nw * _K == e
  steps = chunks // group
  assert steps * group == chunks
  np_ = _pad_rows(n)
  rows_pt = np_ // _NS

  mesh = plsc.VectorSubcoreMesh(core_axis_name="c", subcore_axis_name="s")

  out_type = [jax.ShapeDtypeStruct((_NC, np_, d), jnp.float32)]
  scratch = [
      pltpu.VMEM((steps, group * _K), jnp.int32),
      pltpu.VMEM((chunks, _K), jnp.int32),
      pltpu.VMEM((group * _K, d), jnp.float32),
      pltpu.VMEM((group * _K, d), jnp.float32),
      pltpu.VMEM_SHARED((np_, d), jnp.float32),
      pltpu.SemaphoreType.DMA,
      pltpu.SemaphoreType.DMA,
  ]
  if with_deg:
    out_type.append(jax.ShapeDtypeStruct((_NC, np_, _DW), jnp.float32))
    scratch += [
        pltpu.VMEM((_K, _DW), jnp.float32),
        pltpu.VMEM_SHARED((np_, _DW), jnp.float32),
        pltpu.SemaphoreType.DMA,
    ]

  @functools.partial(
      pl.kernel, mesh=mesh, out_type=out_type, scratch_types=scratch,
      compiler_params=pltpu.CompilerParams(use_tc_tiling_on_sc=False))
  def body(table, esrc, edst, zacc, zdeg, ones, *refs):
    if with_deg:
      (agg_out, deg_out, srcs_v, dsts_v, rows0_v, rows1_v, acc_sh,
       gsem0, gsem1, ones_v, deg_sh, osem) = refs
    else:
      (agg_out, srcs_v, dsts_v, rows0_v, rows1_v, acc_sh, gsem0, gsem1) = refs
    bufs = ((rows0_v, gsem0, None), (rows1_v, gsem1, None))
    cid = lax.axis_index("c")
    sid = lax.axis_index("s")
    wid = cid * _NS + sid

    r0 = sid * rows_pt
    pltpu.sync_copy(esrc.at[wid], srcs_v)
    pltpu.sync_copy(edst.at[wid], dsts_v)
    pltpu.sync_copy(zacc, acc_sh.at[pl.ds(r0, rows_pt)])
    if with_deg:
      pltpu.sync_copy(zdeg, deg_sh.at[pl.ds(r0, rows_pt)])
      pltpu.sync_copy(ones, ones_v)
    plsc.subcore_barrier()

    def gather(j, b):
      rows_v, gsem, _ = bufs[b]
      pltpu.make_async_copy(table.at[srcs_v.at[j]], rows_v, gsem).start()

    def wait_scatter(j, b):
      rows_v, gsem, _ = bufs[b]
      pltpu.make_async_copy(table.at[srcs_v.at[j]], rows_v, gsem).wait()
      for k in range(group):
        rv = rows_v if group == 1 else rows_v.at[pl.ds(_K * k, _K)]
        pltpu.sync_copy(rv, acc_sh.at[dsts_v.at[group * j + k]], add=True)
        if with_deg:
          pltpu.async_copy(ones_v, deg_sh.at[dsts_v.at[group * j + k]], osem,
                           add=True)

    gather(0, 0)

    def pair(i, carry):
      j0 = 2 * i
      gather(j0 + 1, 1)
      wait_scatter(j0, 0)

      @pl.when(j0 + 2 < steps)
      def _():
        gather(j0 + 2, 0)

      wait_scatter(j0 + 1, 1)
      return carry

    lax.fori_loop(0, steps // 2, pair, 0)
    if steps % 2:
      wait_scatter(steps - 1, 0)
    if with_deg:
      def drain(j, carry):
        pltpu.make_async_copy(ones_v, deg_sh.at[dsts_v.at[j]], osem).wait()
        return carry
      lax.fori_loop(0, chunks, drain, 0)
    plsc.subcore_barrier()

    pltpu.sync_copy(acc_sh.at[pl.ds(r0, rows_pt)],
                    agg_out.at[cid, pl.ds(r0, rows_pt)])
    if with_deg:
      pltpu.sync_copy(deg_sh.at[pl.ds(r0, rows_pt)],
                      deg_out.at[cid, pl.ds(r0, rows_pt)])

  return body


def _tc_layer1(x, agg, deg, w_self1, w_neigh1, b1, w_self2, w_neigh2, b2):
  n, d = x.shape
  h = w_self1.shape[0]
  c = w_self2.shape[0]
  bn = 1000
  grid = (n // bn,)

  def tcb(x_b, agg_b, deg_b, ws1, wn1, b1_b, ws2, wn2, b2_b,
          h1_b, h1r_b, p2_b, hs2_b, rdeg_b):
    degs = jnp.maximum(deg_b[0, :, 0] + deg_b[1, :, 0], 1.0)
    rdeg = 1.0 / degs
    rdeg_b[...] = jnp.broadcast_to(rdeg[:, None], rdeg_b.shape)
    mean = (agg_b[0] + agg_b[1]) * rdeg[:, None]
    dn = (((1,), (1,)), ((), ()))
    h1 = (lax.dot_general(x_b[...], ws1[...], dn,
                          preferred_element_type=jnp.float32)
          + lax.dot_general(mean, wn1[...], dn,
                            preferred_element_type=jnp.float32)
          + b1_b[...])
    h1_b[...] = h1
    h1r = jnp.maximum(h1, 0.0)
    h1r_b[...] = h1r
    p2_b[...] = lax.dot_general(h1r, wn2[...], dn,
                                preferred_element_type=jnp.float32)
    hs2_b[...] = lax.dot_general(h1r, ws2[...], dn,
                                 preferred_element_type=jnp.float32) + b2_b[...]

  return pl.pallas_call(
      tcb,
      grid=grid,
      in_specs=[
          pl.BlockSpec((bn, d), lambda i: (i, 0)),
          pl.BlockSpec((_NC, bn, d), lambda i: (0, i, 0)),
          pl.BlockSpec((_NC, bn, _DW), lambda i: (0, i, 0)),
          pl.BlockSpec((h, d), lambda i: (0, 0)),
          pl.BlockSpec((h, d), lambda i: (0, 0)),
          pl.BlockSpec((1, h), lambda i: (0, 0)),
          pl.BlockSpec((c, h), lambda i: (0, 0)),
          pl.BlockSpec((c, h), lambda i: (0, 0)),
          pl.BlockSpec((1, c), lambda i: (0, 0)),
      ],
      out_specs=[
          pl.BlockSpec((bn, h), lambda i: (i, 0)),
          pl.BlockSpec((bn, h), lambda i: (i, 0)),
          pl.BlockSpec((bn, c), lambda i: (i, 0)),
          pl.BlockSpec((bn, c), lambda i: (i, 0)),
          pl.BlockSpec((bn, 16), lambda i: (i, 0)),
      ],
      out_shape=[
          jax.ShapeDtypeStruct((n, h), jnp.float32),
          jax.ShapeDtypeStruct((n, h), jnp.float32),
          jax.ShapeDtypeStruct((n, c), jnp.float32),
          jax.ShapeDtypeStruct((n, c), jnp.float32),
          jax.ShapeDtypeStruct((n, 16), jnp.float32),
      ],
  )(x, agg, deg, w_self1, w_neigh1, b1.reshape(1, h), w_self2, w_neigh2,
    b2.reshape(1, c))


def _sc_layer2_epilogue(n, c, np_):
  nw = _NC * _NS
  main = (n // (nw * 8)) * 8
  extra = n - nw * main
  assert extra % 8 == 0 and extra >= 0
  buf = main + extra

  mesh = plsc.VectorSubcoreMesh(core_axis_name="c", subcore_axis_name="s")

  @functools.partial(
      pl.kernel, mesh=mesh,
      out_type=jax.ShapeDtypeStruct((n, c), jnp.float32),
      scratch_types=[
          pltpu.VMEM((buf, c), jnp.float32),
          pltpu.VMEM((buf, 16), jnp.float32),
          pltpu.VMEM((buf, c), jnp.float32),
          pltpu.VMEM((buf, c), jnp.float32),
          pltpu.VMEM((buf, c), jnp.float32),
      ],
      compiler_params=pltpu.CompilerParams(use_tc_tiling_on_sc=False))
  def body(hs2b, rdeg, agg2, h2, hb_v, rd_v, a0_v, a1_v, out_v):
    cid = lax.axis_index("c")
    sid = lax.axis_index("s")
    wid = cid * _NS + sid
    t0 = wid * main
    last = wid == nw - 1

    pltpu.sync_copy(hs2b.at[pl.ds(t0, main)], hb_v.at[pl.ds(0, main)])
    pltpu.sync_copy(rdeg.at[pl.ds(t0, main)], rd_v.at[pl.ds(0, main)])
    pltpu.sync_copy(agg2.at[0, pl.ds(t0, main)], a0_v.at[pl.ds(0, main)])
    pltpu.sync_copy(agg2.at[1, pl.ds(t0, main)], a1_v.at[pl.ds(0, main)])
    if extra:
      t1 = nw * main

      @pl.when(last)
      def _():
        pltpu.sync_copy(hs2b.at[pl.ds(t1, extra)], hb_v.at[pl.ds(main, extra)])
        pltpu.sync_copy(rdeg.at[pl.ds(t1, extra)], rd_v.at[pl.ds(main, extra)])
        pltpu.sync_copy(agg2.at[0, pl.ds(t1, extra)],
                        a0_v.at[pl.ds(main, extra)])
        pltpu.sync_copy(agg2.at[1, pl.ds(t1, extra)],
                        a1_v.at[pl.ds(main, extra)])

    nrows = jnp.where(last, main + extra, main)

    def row(r, carry):
      out_v[r] = hb_v[r] + (a0_v[r] + a1_v[r]) * rd_v[r]
      return carry

    lax.fori_loop(0, nrows, row, 0)

    pltpu.sync_copy(out_v.at[pl.ds(0, main)], h2.at[pl.ds(t0, main)])
    if extra:
      t1 = nw * main

      @pl.when(last)
      def _():
        pltpu.sync_copy(out_v.at[pl.ds(main, extra)], h2.at[pl.ds(t1, extra)])

  return body


def _tc_layer2(hs2, agg2, deg, b2):
  n, c = hs2.shape

  def tcc(hs2_b, agg2_b, deg_b, b2_b, h2_b):
    degs = jnp.maximum(deg_b[0, :, 0] + deg_b[1, :, 0], 1.0)
    h2_b[...] = hs2_b[...] + (agg2_b[0] + agg2_b[1]) / degs[:, None] + b2_b[...]

  return pl.pallas_call(
      tcc,
      grid=(1,),
      in_specs=[
          pl.BlockSpec((n, c), lambda i: (0, 0)),
          pl.BlockSpec((_NC, n, c), lambda i: (0, 0, 0)),
          pl.BlockSpec((_NC, n, _DW), lambda i: (0, 0, 0)),
          pl.BlockSpec((1, c), lambda i: (0, 0)),
      ],
      out_specs=pl.BlockSpec((n, c), lambda i: (0, 0)),
      out_shape=jax.ShapeDtypeStruct((n, c), jnp.float32),
  )(hs2, agg2, deg, b2.reshape(1, c))


def kernel(x, edge_index, W_self1, W_neigh1, b1, W_self2, W_neigh2, b2):
  n, d = x.shape
  e = edge_index.shape[1]
  c = W_self2.shape[0]
  rows_pt = _pad_rows(n) // _NS

  nw = _NC * _NS
  chunks = e // (nw * _K)
  esrc1 = edge_index[0].reshape(nw, chunks, _K)
  esrc2 = edge_index[0].reshape(nw, chunks // _G2, _G2 * _K)
  edst = edge_index[1].reshape(nw, chunks, _K)
  zacc = jnp.zeros((rows_pt, d), jnp.float32)
  zdeg = jnp.zeros((rows_pt, _DW), jnp.float32)
  zacc2 = jnp.zeros((rows_pt, c), jnp.float32)
  ones = jnp.ones((_K, _DW), jnp.float32)

  agg1, deg = _make_sc_agg(n, e, d, True)(x, esrc1, edst, zacc, zdeg, ones)
  h1, h1r, p2, hs2b, rdeg = _tc_layer1(x, agg1, deg, W_self1, W_neigh1, b1,
                                       W_self2, W_neigh2, b2)
  agg2 = _make_sc_agg(n, e, c, False, group=_G2)(
      p2, esrc2, edst, zacc2, zdeg, ones)[0]
  h2 = _sc_layer2_epilogue(n, c, _pad_rows(n))(hs2b, rdeg, agg2)
  return (h2, h1, h1r)


# speedup vs baseline: 1.4192x; 1.0143x over previous
import functools

import jax
import jax.numpy as jnp
from jax import lax
from jax.experimental import pallas as pl
from jax.experimental.pallas import tpu as pltpu
from jax.experimental.pallas import tpu_sc as plsc

_NC = 2
_NS = 16
_K = 80
_DW = 8
_G2 = 5


def _pad_rows(n):
  q = _NS * 8
  return ((n + q - 1) // q) * q


def _make_sc_agg(n, e, d, with_deg, group=1):
  nw = _NC * _NS
  chunks = e // (nw * _K)
  assert chunks * nw * _K == e
  steps = chunks // group
  assert steps * group == chunks
  np_ = _pad_rows(n)
  rows_pt = np_ // _NS

  mesh = plsc.VectorSubcoreMesh(core_axis_name="c", subcore_axis_name="s")

  out_type = [jax.ShapeDtypeStruct((_NC, np_, d), jnp.float32)]
  scratch = [
      pltpu.VMEM((steps, group * _K), jnp.int32),
      pltpu.VMEM((chunks, _K), jnp.int32),
      pltpu.VMEM((group * _K, d), jnp.float32),
      pltpu.VMEM((group * _K, d), jnp.float32),
      pltpu.VMEM_SHARED((np_, d), jnp.float32),
      pltpu.SemaphoreType.DMA,
      pltpu.SemaphoreType.DMA,
  ]
  if with_deg:
    out_type.append(jax.ShapeDtypeStruct((_NC, np_, _DW), jnp.float32))
    scratch += [
        pltpu.VMEM((_K, _DW), jnp.float32),
        pltpu.VMEM_SHARED((np_, _DW), jnp.float32),
        pltpu.SemaphoreType.DMA,
    ]

  @functools.partial(
      pl.kernel, mesh=mesh, out_type=out_type, scratch_types=scratch,
      compiler_params=pltpu.CompilerParams(use_tc_tiling_on_sc=False))
  def body(table, esrc, edst, zacc, zdeg, ones, *refs):
    if with_deg:
      (agg_out, deg_out, srcs_v, dsts_v, rows0_v, rows1_v, acc_sh,
       gsem0, gsem1, ones_v, deg_sh, osem) = refs
    else:
      (agg_out, srcs_v, dsts_v, rows0_v, rows1_v, acc_sh, gsem0, gsem1) = refs
    bufs = ((rows0_v, gsem0, None), (rows1_v, gsem1, None))
    cid = lax.axis_index("c")
    sid = lax.axis_index("s")
    wid = cid * _NS + sid

    r0 = sid * rows_pt
    init = [(esrc.at[wid], srcs_v), (edst.at[wid], dsts_v),
            (zacc, acc_sh.at[pl.ds(r0, rows_pt)])]
    if with_deg:
      init += [(zdeg, deg_sh.at[pl.ds(r0, rows_pt)]), (ones, ones_v)]
    for src, dst in init:
      pltpu.make_async_copy(src, dst, gsem0).start()
    for src, dst in init:
      pltpu.make_async_copy(src, dst, gsem0).wait()
    plsc.subcore_barrier()

    def gather(j, b):
      rows_v, gsem, _ = bufs[b]
      pltpu.make_async_copy(table.at[srcs_v.at[j]], rows_v, gsem).start()

    def wait_scatter(j, b):
      rows_v, gsem, _ = bufs[b]
      pltpu.make_async_copy(table.at[srcs_v.at[j]], rows_v, gsem).wait()
      for k in range(group):
        rv = rows_v if group == 1 else rows_v.at[pl.ds(_K * k, _K)]
        pltpu.sync_copy(rv, acc_sh.at[dsts_v.at[group * j + k]], add=True)
        if with_deg:
          pltpu.async_copy(ones_v, deg_sh.at[dsts_v.at[group * j + k]], osem,
                           add=True)

    gather(0, 0)

    def pair(i, carry):
      j0 = 2 * i
      gather(j0 + 1, 1)
      wait_scatter(j0, 0)

      @pl.when(j0 + 2 < steps)
      def _():
        gather(j0 + 2, 0)

      wait_scatter(j0 + 1, 1)
      return carry

    lax.fori_loop(0, steps // 2, pair, 0)
    if steps % 2:
      wait_scatter(steps - 1, 0)
    if with_deg:
      def drain(j, carry):
        pltpu.make_async_copy(ones_v, deg_sh.at[dsts_v.at[j]], osem).wait()
        return carry
      lax.fori_loop(0, chunks, drain, 0)
    plsc.subcore_barrier()

    pltpu.sync_copy(acc_sh.at[pl.ds(r0, rows_pt)],
                    agg_out.at[cid, pl.ds(r0, rows_pt)])
    if with_deg:
      pltpu.sync_copy(deg_sh.at[pl.ds(r0, rows_pt)],
                      deg_out.at[cid, pl.ds(r0, rows_pt)])

  return body


def _tc_layer1(x, agg, deg, w_self1, w_neigh1, b1, w_self2, w_neigh2, b2):
  n, d = x.shape
  h = w_self1.shape[0]
  c = w_self2.shape[0]
  bn = 1000
  grid = (n // bn,)

  def tcb(x_b, agg_b, deg_b, ws1, wn1, b1_b, ws2, wn2, b2_b,
          h1_b, h1r_b, p2_b, hs2_b, rdeg_b):
    degs = jnp.maximum(deg_b[0, :, 0] + deg_b[1, :, 0], 1.0)
    rdeg = 1.0 / degs
    rdeg_b[...] = jnp.broadcast_to(rdeg[:, None], rdeg_b.shape)
    mean = (agg_b[0] + agg_b[1]) * rdeg[:, None]
    dn = (((1,), (1,)), ((), ()))
    h1 = (lax.dot_general(x_b[...], ws1[...], dn,
                          preferred_element_type=jnp.float32)
          + lax.dot_general(mean, wn1[...], dn,
                            preferred_element_type=jnp.float32)
          + b1_b[...])
    h1_b[...] = h1
    h1r = jnp.maximum(h1, 0.0)
    h1r_b[...] = h1r
    p2_b[...] = lax.dot_general(h1r, wn2[...], dn,
                                preferred_element_type=jnp.float32)
    hs2_b[...] = lax.dot_general(h1r, ws2[...], dn,
                                 preferred_element_type=jnp.float32) + b2_b[...]

  return pl.pallas_call(
      tcb,
      grid=grid,
      in_specs=[
          pl.BlockSpec((bn, d), lambda i: (i, 0)),
          pl.BlockSpec((_NC, bn, d), lambda i: (0, i, 0)),
          pl.BlockSpec((_NC, bn, _DW), lambda i: (0, i, 0)),
          pl.BlockSpec((h, d), lambda i: (0, 0)),
          pl.BlockSpec((h, d), lambda i: (0, 0)),
          pl.BlockSpec((1, h), lambda i: (0, 0)),
          pl.BlockSpec((c, h), lambda i: (0, 0)),
          pl.BlockSpec((c, h), lambda i: (0, 0)),
          pl.BlockSpec((1, c), lambda i: (0, 0)),
      ],
      out_specs=[
          pl.BlockSpec((bn, h), lambda i: (i, 0)),
          pl.BlockSpec((bn, h), lambda i: (i, 0)),
          pl.BlockSpec((bn, c), lambda i: (i, 0)),
          pl.BlockSpec((bn, c), lambda i: (i, 0)),
          pl.BlockSpec((bn, 16), lambda i: (i, 0)),
      ],
      out_shape=[
          jax.ShapeDtypeStruct((n, h), jnp.float32),
          jax.ShapeDtypeStruct((n, h), jnp.float32),
          jax.ShapeDtypeStruct((n, c), jnp.float32),
          jax.ShapeDtypeStruct((n, c), jnp.float32),
          jax.ShapeDtypeStruct((n, 16), jnp.float32),
      ],
  )(x, agg, deg, w_self1, w_neigh1, b1.reshape(1, h), w_self2, w_neigh2,
    b2.reshape(1, c))


def _sc_layer2_epilogue(n, c, np_):
  nw = _NC * _NS
  main = (n // (nw * 8)) * 8
  extra = n - nw * main
  assert extra % 8 == 0 and extra >= 0
  buf = main + extra

  mesh = plsc.VectorSubcoreMesh(core_axis_name="c", subcore_axis_name="s")

  @functools.partial(
      pl.kernel, mesh=mesh,
      out_type=jax.ShapeDtypeStruct((n, c), jnp.float32),
      scratch_types=[
          pltpu.VMEM((buf, c), jnp.float32),
          pltpu.VMEM((buf, 16), jnp.float32),
          pltpu.VMEM((buf, c), jnp.float32),
          pltpu.VMEM((buf, c), jnp.float32),
          pltpu.VMEM((buf, c), jnp.float32),
      ],
      compiler_params=pltpu.CompilerParams(use_tc_tiling_on_sc=False))
  def body(hs2b, rdeg, agg2, h2, hb_v, rd_v, a0_v, a1_v, out_v):
    cid = lax.axis_index("c")
    sid = lax.axis_index("s")
    wid = cid * _NS + sid
    t0 = wid * main
    last = wid == nw - 1

    pltpu.sync_copy(hs2b.at[pl.ds(t0, main)], hb_v.at[pl.ds(0, main)])
    pltpu.sync_copy(rdeg.at[pl.ds(t0, main)], rd_v.at[pl.ds(0, main)])
    pltpu.sync_copy(agg2.at[0, pl.ds(t0, main)], a0_v.at[pl.ds(0, main)])
    pltpu.sync_copy(agg2.at[1, pl.ds(t0, main)], a1_v.at[pl.ds(0, main)])
    if extra:
      t1 = nw * main

      @pl.when(last)
      def _():
        pltpu.sync_copy(hs2b.at[pl.ds(t1, extra)], hb_v.at[pl.ds(main, extra)])
        pltpu.sync_copy(rdeg.at[pl.ds(t1, extra)], rd_v.at[pl.ds(main, extra)])
        pltpu.sync_copy(agg2.at[0, pl.ds(t1, extra)],
                        a0_v.at[pl.ds(main, extra)])
        pltpu.sync_copy(agg2.at[1, pl.ds(t1, extra)],
                        a1_v.at[pl.ds(main, extra)])

    nrows = jnp.where(last, main + extra, main)

    def row(r, carry):
      out_v[r] = hb_v[r] + (a0_v[r] + a1_v[r]) * rd_v[r]
      return carry

    lax.fori_loop(0, nrows, row, 0)

    pltpu.sync_copy(out_v.at[pl.ds(0, main)], h2.at[pl.ds(t0, main)])
    if extra:
      t1 = nw * main

      @pl.when(last)
      def _():
        pltpu.sync_copy(out_v.at[pl.ds(main, extra)], h2.at[pl.ds(t1, extra)])

  return body


def _tc_layer2(hs2, agg2, deg, b2):
  n, c = hs2.shape

  def tcc(hs2_b, agg2_b, deg_b, b2_b, h2_b):
    degs = jnp.maximum(deg_b[0, :, 0] + deg_b[1, :, 0], 1.0)
    h2_b[...] = hs2_b[...] + (agg2_b[0] + agg2_b[1]) / degs[:, None] + b2_b[...]

  return pl.pallas_call(
      tcc,
      grid=(1,),
      in_specs=[
          pl.BlockSpec((n, c), lambda i: (0, 0)),
          pl.BlockSpec((_NC, n, c), lambda i: (0, 0, 0)),
          pl.BlockSpec((_NC, n, _DW), lambda i: (0, 0, 0)),
          pl.BlockSpec((1, c), lambda i: (0, 0)),
      ],
      out_specs=pl.BlockSpec((n, c), lambda i: (0, 0)),
      out_shape=jax.ShapeDtypeStruct((n, c), jnp.float32),
  )(hs2, agg2, deg, b2.reshape(1, c))


def kernel(x, edge_index, W_self1, W_neigh1, b1, W_self2, W_neigh2, b2):
  n, d = x.shape
  e = edge_index.shape[1]
  c = W_self2.shape[0]
  rows_pt = _pad_rows(n) // _NS

  nw = _NC * _NS
  chunks = e // (nw * _K)
  esrc1 = edge_index[0].reshape(nw, chunks, _K)
  esrc2 = edge_index[0].reshape(nw, chunks // _G2, _G2 * _K)
  edst = edge_index[1].reshape(nw, chunks, _K)
  zacc = jnp.zeros((rows_pt, d), jnp.float32)
  zdeg = jnp.zeros((rows_pt, _DW), jnp.float32)
  zacc2 = jnp.zeros((rows_pt, c), jnp.float32)
  ones = jnp.ones((_K, _DW), jnp.float32)

  agg1, deg = _make_sc_agg(n, e, d, True)(x, esrc1, edst, zacc, zdeg, ones)
  h1, h1r, p2, hs2b, rdeg = _tc_layer1(x, agg1, deg, W_self1, W_neigh1, b1,
                                       W_self2, W_neigh2, b2)
  agg2 = _make_sc_agg(n, e, c, False, group=_G2)(
      p2, esrc2, edst, zacc2, zdeg, ones)[0]
  h2 = _sc_layer2_epilogue(n, c, _pad_rows(n))(hs2b, rdeg, agg2)
  return (h2, h1, h1r)


# speedup vs baseline: 1.4372x; 1.0127x over previous
import functools

import jax
import jax.numpy as jnp
from jax import lax
from jax.experimental import pallas as pl
from jax.experimental.pallas import tpu as pltpu
from jax.experimental.pallas import tpu_sc as plsc

_NC = 2
_NS = 16
_K = 80
_DW = 8
_G2 = 5


def _pad_rows(n):
  q = _NS * 8
  return ((n + q - 1) // q) * q


def _make_sc_agg(n, e, d, with_deg, group=1):
  nw = _NC * _NS
  chunks = e // (nw * _K)
  assert chunks * nw * _K == e
  steps = chunks // group
  assert steps * group == chunks
  np_ = _pad_rows(n)
  rows_pt = np_ // _NS

  mesh = plsc.VectorSubcoreMesh(core_axis_name="c", subcore_axis_name="s")

  out_type = [jax.ShapeDtypeStruct((_NC, np_, d), jnp.float32)]
  scratch = [
      pltpu.VMEM((steps, group * _K), jnp.int32),
      pltpu.VMEM((chunks, _K), jnp.int32),
      pltpu.VMEM((group * _K, d), jnp.float32),
      pltpu.VMEM((group * _K, d), jnp.float32),
      pltpu.VMEM_SHARED((np_, d), jnp.float32),
      pltpu.SemaphoreType.DMA,
      pltpu.SemaphoreType.DMA,
  ]
  if with_deg:
    out_type.append(jax.ShapeDtypeStruct((_NC, np_, _DW), jnp.float32))
    scratch += [
        pltpu.VMEM((_K, _DW), jnp.float32),
        pltpu.VMEM_SHARED((np_, _DW), jnp.float32),
        pltpu.SemaphoreType.DMA,
    ]

  @functools.partial(
      pl.kernel, mesh=mesh, out_type=out_type, scratch_types=scratch,
      compiler_params=pltpu.CompilerParams(use_tc_tiling_on_sc=False))
  def body(table, esrc, edst, zacc, zdeg, ones, *refs):
    if with_deg:
      (agg_out, deg_out, srcs_v, dsts_v, rows0_v, rows1_v, acc_sh,
       gsem0, gsem1, ones_v, deg_sh, osem) = refs
    else:
      (agg_out, srcs_v, dsts_v, rows0_v, rows1_v, acc_sh, gsem0, gsem1) = refs
    bufs = ((rows0_v, gsem0, None), (rows1_v, gsem1, None))
    cid = lax.axis_index("c")
    sid = lax.axis_index("s")
    wid = cid * _NS + sid

    r0 = sid * rows_pt
    init = [(esrc.at[wid], srcs_v), (edst.at[wid], dsts_v),
            (zacc, acc_sh.at[pl.ds(r0, rows_pt)])]
    if with_deg:
      init += [(zdeg, deg_sh.at[pl.ds(r0, rows_pt)]), (ones, ones_v)]
    for src, dst in init:
      pltpu.make_async_copy(src, dst, gsem0).start()
    for src, dst in init:
      pltpu.make_async_copy(src, dst, gsem0).wait()
    plsc.subcore_barrier()

    def gather(j, b):
      rows_v, gsem, _ = bufs[b]
      pltpu.make_async_copy(table.at[srcs_v.at[j]], rows_v, gsem).start()

    def wait_scatter(j, b):
      rows_v, gsem, _ = bufs[b]
      pltpu.make_async_copy(table.at[srcs_v.at[j]], rows_v, gsem).wait()
      for k in range(group):
        rv = rows_v if group == 1 else rows_v.at[pl.ds(_K * k, _K)]
        pltpu.sync_copy(rv, acc_sh.at[dsts_v.at[group * j + k]], add=True)
        if with_deg:
          pltpu.async_copy(ones_v, deg_sh.at[dsts_v.at[group * j + k]], osem,
                           add=True)

    gather(0, 0)

    def pair(i, carry):
      j0 = 2 * i
      gather(j0 + 1, 1)
      wait_scatter(j0, 0)

      @pl.when(j0 + 2 < steps)
      def _():
        gather(j0 + 2, 0)

      wait_scatter(j0 + 1, 1)
      return carry

    lax.fori_loop(0, steps // 2, pair, 0)
    if steps % 2:
      wait_scatter(steps - 1, 0)
    if with_deg:
      def drain(j, carry):
        pltpu.make_async_copy(ones_v, deg_sh.at[dsts_v.at[j]], osem).wait()
        return carry
      lax.fori_loop(0, chunks, drain, 0)
    plsc.subcore_barrier()

    pltpu.sync_copy(acc_sh.at[pl.ds(r0, rows_pt)],
                    agg_out.at[cid, pl.ds(r0, rows_pt)])
    if with_deg:
      pltpu.sync_copy(deg_sh.at[pl.ds(r0, rows_pt)],
                      deg_out.at[cid, pl.ds(r0, rows_pt)])

  return body


def _tc_layer1(x, agg, deg, w_self1, w_neigh1, b1, w_self2, w_neigh2, b2):
  n, d = x.shape
  h = w_self1.shape[0]
  c = w_self2.shape[0]
  bn = 2000
  grid = (n // bn,)

  def tcb(x_b, agg_b, deg_b, ws1, wn1, b1_b, ws2, wn2, b2_b,
          h1_b, h1r_b, p2_b, hs2_b, rdeg_b):
    degs = jnp.maximum(deg_b[0, :, 0] + deg_b[1, :, 0], 1.0)
    rdeg = 1.0 / degs
    rdeg_b[...] = jnp.broadcast_to(rdeg[:, None], rdeg_b.shape)
    mean = (agg_b[0] + agg_b[1]) * rdeg[:, None]
    dn = (((1,), (1,)), ((), ()))
    h1 = (lax.dot_general(x_b[...], ws1[...], dn,
                          preferred_element_type=jnp.float32)
          + lax.dot_general(mean, wn1[...], dn,
                            preferred_element_type=jnp.float32)
          + b1_b[...])
    h1_b[...] = h1
    h1r = jnp.maximum(h1, 0.0)
    h1r_b[...] = h1r
    p2_b[...] = lax.dot_general(h1r, wn2[...], dn,
                                preferred_element_type=jnp.float32)
    hs2_b[...] = lax.dot_general(h1r, ws2[...], dn,
                                 preferred_element_type=jnp.float32) + b2_b[...]

  return pl.pallas_call(
      tcb,
      grid=grid,
      in_specs=[
          pl.BlockSpec((bn, d), lambda i: (i, 0)),
          pl.BlockSpec((_NC, bn, d), lambda i: (0, i, 0)),
          pl.BlockSpec((_NC, bn, _DW), lambda i: (0, i, 0)),
          pl.BlockSpec((h, d), lambda i: (0, 0)),
          pl.BlockSpec((h, d), lambda i: (0, 0)),
          pl.BlockSpec((1, h), lambda i: (0, 0)),
          pl.BlockSpec((c, h), lambda i: (0, 0)),
          pl.BlockSpec((c, h), lambda i: (0, 0)),
          pl.BlockSpec((1, c), lambda i: (0, 0)),
      ],
      out_specs=[
          pl.BlockSpec((bn, h), lambda i: (i, 0)),
          pl.BlockSpec((bn, h), lambda i: (i, 0)),
          pl.BlockSpec((bn, c), lambda i: (i, 0)),
          pl.BlockSpec((bn, c), lambda i: (i, 0)),
          pl.BlockSpec((bn, 16), lambda i: (i, 0)),
      ],
      out_shape=[
          jax.ShapeDtypeStruct((n, h), jnp.float32),
          jax.ShapeDtypeStruct((n, h), jnp.float32),
          jax.ShapeDtypeStruct((n, c), jnp.float32),
          jax.ShapeDtypeStruct((n, c), jnp.float32),
          jax.ShapeDtypeStruct((n, 16), jnp.float32),
      ],
  )(x, agg, deg, w_self1, w_neigh1, b1.reshape(1, h), w_self2, w_neigh2,
    b2.reshape(1, c))


def _sc_layer2_epilogue(n, c, np_):
  nw = _NC * _NS
  main = (n // (nw * 8)) * 8
  extra = n - nw * main
  assert extra % 8 == 0 and extra >= 0
  buf = main + extra

  mesh = plsc.VectorSubcoreMesh(core_axis_name="c", subcore_axis_name="s")

  @functools.partial(
      pl.kernel, mesh=mesh,
      out_type=jax.ShapeDtypeStruct((n, c), jnp.float32),
      scratch_types=[
          pltpu.VMEM((buf, c), jnp.float32),
          pltpu.VMEM((buf, 16), jnp.float32),
          pltpu.VMEM((buf, c), jnp.float32),
          pltpu.VMEM((buf, c), jnp.float32),
          pltpu.VMEM((buf, c), jnp.float32),
      ],
      compiler_params=pltpu.CompilerParams(use_tc_tiling_on_sc=False))
  def body(hs2b, rdeg, agg2, h2, hb_v, rd_v, a0_v, a1_v, out_v):
    cid = lax.axis_index("c")
    sid = lax.axis_index("s")
    wid = cid * _NS + sid
    t0 = wid * main
    last = wid == nw - 1

    pltpu.sync_copy(hs2b.at[pl.ds(t0, main)], hb_v.at[pl.ds(0, main)])
    pltpu.sync_copy(rdeg.at[pl.ds(t0, main)], rd_v.at[pl.ds(0, main)])
    pltpu.sync_copy(agg2.at[0, pl.ds(t0, main)], a0_v.at[pl.ds(0, main)])
    pltpu.sync_copy(agg2.at[1, pl.ds(t0, main)], a1_v.at[pl.ds(0, main)])
    if extra:
      t1 = nw * main

      @pl.when(last)
      def _():
        pltpu.sync_copy(hs2b.at[pl.ds(t1, extra)], hb_v.at[pl.ds(main, extra)])
        pltpu.sync_copy(rdeg.at[pl.ds(t1, extra)], rd_v.at[pl.ds(main, extra)])
        pltpu.sync_copy(agg2.at[0, pl.ds(t1, extra)],
                        a0_v.at[pl.ds(main, extra)])
        pltpu.sync_copy(agg2.at[1, pl.ds(t1, extra)],
                        a1_v.at[pl.ds(main, extra)])

    nrows = jnp.where(last, main + extra, main)

    def row(r, carry):
      out_v[r] = hb_v[r] + (a0_v[r] + a1_v[r]) * rd_v[r]
      return carry

    lax.fori_loop(0, nrows, row, 0)

    pltpu.sync_copy(out_v.at[pl.ds(0, main)], h2.at[pl.ds(t0, main)])
    if extra:
      t1 = nw * main

      @pl.when(last)
      def _():
        pltpu.sync_copy(out_v.at[pl.ds(main, extra)], h2.at[pl.ds(t1, extra)])

  return body


def _tc_layer2(hs2, agg2, deg, b2):
  n, c = hs2.shape

  def tcc(hs2_b, agg2_b, deg_b, b2_b, h2_b):
    degs = jnp.maximum(deg_b[0, :, 0] + deg_b[1, :, 0], 1.0)
    h2_b[...] = hs2_b[...] + (agg2_b[0] + agg2_b[1]) / degs[:, None] + b2_b[...]

  return pl.pallas_call(
      tcc,
      grid=(1,),
      in_specs=[
          pl.BlockSpec((n, c), lambda i: (0, 0)),
          pl.BlockSpec((_NC, n, c), lambda i: (0, 0, 0)),
          pl.BlockSpec((_NC, n, _DW), lambda i: (0, 0, 0)),
          pl.BlockSpec((1, c), lambda i: (0, 0)),
      ],
      out_specs=pl.BlockSpec((n, c), lambda i: (0, 0)),
      out_shape=jax.ShapeDtypeStruct((n, c), jnp.float32),
  )(hs2, agg2, deg, b2.reshape(1, c))


def kernel(x, edge_index, W_self1, W_neigh1, b1, W_self2, W_neigh2, b2):
  n, d = x.shape
  e = edge_index.shape[1]
  c = W_self2.shape[0]
  rows_pt = _pad_rows(n) // _NS

  nw = _NC * _NS
  chunks = e // (nw * _K)
  esrc1 = edge_index[0].reshape(nw, chunks, _K)
  esrc2 = edge_index[0].reshape(nw, chunks // _G2, _G2 * _K)
  edst = edge_index[1].reshape(nw, chunks, _K)
  zacc = jnp.zeros((rows_pt, d), jnp.float32)
  zdeg = jnp.zeros((rows_pt, _DW), jnp.float32)
  zacc2 = jnp.zeros((rows_pt, c), jnp.float32)
  ones = jnp.ones((_K, _DW), jnp.float32)

  agg1, deg = _make_sc_agg(n, e, d, True)(x, esrc1, edst, zacc, zdeg, ones)
  h1, h1r, p2, hs2b, rdeg = _tc_layer1(x, agg1, deg, W_self1, W_neigh1, b1,
                                       W_self2, W_neigh2, b2)
  agg2 = _make_sc_agg(n, e, c, False, group=_G2)(
      p2, esrc2, edst, zacc2, zdeg, ones)[0]
  h2 = _sc_layer2_epilogue(n, c, _pad_rows(n))(hs2b, rdeg, agg2)
  return (h2, h1, h1r)


# speedup vs baseline: 1.4519x; 1.0102x over previous
import functools

import jax
import jax.numpy as jnp
from jax import lax
from jax.experimental import pallas as pl
from jax.experimental.pallas import tpu as pltpu
from jax.experimental.pallas import tpu_sc as plsc

_NC = 2
_NS = 16
_K = 80
_DW = 8
_G2 = 25


def _pad_rows(n):
  q = _NS * 8
  return ((n + q - 1) // q) * q


def _make_sc_agg(n, e, d, with_deg, group=1):
  nw = _NC * _NS
  chunks = e // (nw * _K)
  assert chunks * nw * _K == e
  steps = chunks // group
  assert steps * group == chunks
  np_ = _pad_rows(n)
  rows_pt = np_ // _NS

  mesh = plsc.VectorSubcoreMesh(core_axis_name="c", subcore_axis_name="s")

  out_type = [jax.ShapeDtypeStruct((_NC, np_, d), jnp.float32)]
  scratch = [
      pltpu.VMEM((steps, group * _K), jnp.int32),
      pltpu.VMEM((chunks, _K), jnp.int32),
      pltpu.VMEM((group * _K, d), jnp.float32),
      pltpu.VMEM((group * _K, d), jnp.float32),
      pltpu.VMEM_SHARED((np_, d), jnp.float32),
      pltpu.SemaphoreType.DMA,
      pltpu.SemaphoreType.DMA,
  ]
  if with_deg:
    out_type.append(jax.ShapeDtypeStruct((_NC, np_, _DW), jnp.float32))
    scratch += [
        pltpu.VMEM((_K, _DW), jnp.float32),
        pltpu.VMEM_SHARED((np_, _DW), jnp.float32),
        pltpu.SemaphoreType.DMA,
    ]

  @functools.partial(
      pl.kernel, mesh=mesh, out_type=out_type, scratch_types=scratch,
      compiler_params=pltpu.CompilerParams(use_tc_tiling_on_sc=False))
  def body(table, esrc, edst, zacc, zdeg, ones, *refs):
    if with_deg:
      (agg_out, deg_out, srcs_v, dsts_v, rows0_v, rows1_v, acc_sh,
       gsem0, gsem1, ones_v, deg_sh, osem) = refs
    else:
      (agg_out, srcs_v, dsts_v, rows0_v, rows1_v, acc_sh, gsem0, gsem1) = refs
    bufs = ((rows0_v, gsem0, None), (rows1_v, gsem1, None))
    cid = lax.axis_index("c")
    sid = lax.axis_index("s")
    wid = cid * _NS + sid

    r0 = sid * rows_pt
    init = [(esrc.at[wid], srcs_v), (edst.at[wid], dsts_v),
            (zacc, acc_sh.at[pl.ds(r0, rows_pt)])]
    if with_deg:
      init += [(zdeg, deg_sh.at[pl.ds(r0, rows_pt)]), (ones, ones_v)]
    for src, dst in init:
      pltpu.make_async_copy(src, dst, gsem0).start()
    for src, dst in init:
      pltpu.make_async_copy(src, dst, gsem0).wait()
    plsc.subcore_barrier()

    def gather(j, b):
      rows_v, gsem, _ = bufs[b]
      pltpu.make_async_copy(table.at[srcs_v.at[j]], rows_v, gsem).start()

    def wait_scatter(j, b):
      rows_v, gsem, _ = bufs[b]
      pltpu.make_async_copy(table.at[srcs_v.at[j]], rows_v, gsem).wait()
      for k in range(group):
        rv = rows_v if group == 1 else rows_v.at[pl.ds(_K * k, _K)]
        pltpu.sync_copy(rv, acc_sh.at[dsts_v.at[group * j + k]], add=True)
        if with_deg:
          pltpu.async_copy(ones_v, deg_sh.at[dsts_v.at[group * j + k]], osem,
                           add=True)

    gather(0, 0)

    def pair(i, carry):
      j0 = 2 * i
      gather(j0 + 1, 1)
      wait_scatter(j0, 0)

      @pl.when(j0 + 2 < steps)
      def _():
        gather(j0 + 2, 0)

      wait_scatter(j0 + 1, 1)
      return carry

    lax.fori_loop(0, steps // 2, pair, 0)
    if steps % 2:
      wait_scatter(steps - 1, 0)
    if with_deg:
      def drain(j, carry):
        pltpu.make_async_copy(ones_v, deg_sh.at[dsts_v.at[j]], osem).wait()
        return carry
      lax.fori_loop(0, chunks, drain, 0)
    plsc.subcore_barrier()

    pltpu.sync_copy(acc_sh.at[pl.ds(r0, rows_pt)],
                    agg_out.at[cid, pl.ds(r0, rows_pt)])
    if with_deg:
      pltpu.sync_copy(deg_sh.at[pl.ds(r0, rows_pt)],
                      deg_out.at[cid, pl.ds(r0, rows_pt)])

  return body


def _tc_layer1(x, agg, deg, w_self1, w_neigh1, b1, w_self2, w_neigh2, b2):
  n, d = x.shape
  h = w_self1.shape[0]
  c = w_self2.shape[0]
  bn = 2000
  grid = (n // bn,)

  def tcb(x_b, agg_b, deg_b, ws1, wn1, b1_b, ws2, wn2, b2_b,
          h1_b, h1r_b, p2_b, hs2_b, rdeg_b):
    degs = jnp.maximum(deg_b[0, :, 0] + deg_b[1, :, 0], 1.0)
    rdeg = 1.0 / degs
    rdeg_b[...] = jnp.broadcast_to(rdeg[:, None], rdeg_b.shape)
    mean = (agg_b[0] + agg_b[1]) * rdeg[:, None]
    dn = (((1,), (1,)), ((), ()))
    h1 = (lax.dot_general(x_b[...], ws1[...], dn,
                          preferred_element_type=jnp.float32)
          + lax.dot_general(mean, wn1[...], dn,
                            preferred_element_type=jnp.float32)
          + b1_b[...])
    h1_b[...] = h1
    h1r = jnp.maximum(h1, 0.0)
    h1r_b[...] = h1r
    p2_b[...] = lax.dot_general(h1r, wn2[...], dn,
                                preferred_element_type=jnp.float32)
    hs2_b[...] = lax.dot_general(h1r, ws2[...], dn,
                                 preferred_element_type=jnp.float32) + b2_b[...]

  return pl.pallas_call(
      tcb,
      grid=grid,
      in_specs=[
          pl.BlockSpec((bn, d), lambda i: (i, 0)),
          pl.BlockSpec((_NC, bn, d), lambda i: (0, i, 0)),
          pl.BlockSpec((_NC, bn, _DW), lambda i: (0, i, 0)),
          pl.BlockSpec((h, d), lambda i: (0, 0)),
          pl.BlockSpec((h, d), lambda i: (0, 0)),
          pl.BlockSpec((1, h), lambda i: (0, 0)),
          pl.BlockSpec((c, h), lambda i: (0, 0)),
          pl.BlockSpec((c, h), lambda i: (0, 0)),
          pl.BlockSpec((1, c), lambda i: (0, 0)),
      ],
      out_specs=[
          pl.BlockSpec((bn, h), lambda i: (i, 0)),
          pl.BlockSpec((bn, h), lambda i: (i, 0)),
          pl.BlockSpec((bn, c), lambda i: (i, 0)),
          pl.BlockSpec((bn, c), lambda i: (i, 0)),
          pl.BlockSpec((bn, 16), lambda i: (i, 0)),
      ],
      out_shape=[
          jax.ShapeDtypeStruct((n, h), jnp.float32),
          jax.ShapeDtypeStruct((n, h), jnp.float32),
          jax.ShapeDtypeStruct((n, c), jnp.float32),
          jax.ShapeDtypeStruct((n, c), jnp.float32),
          jax.ShapeDtypeStruct((n, 16), jnp.float32),
      ],
  )(x, agg, deg, w_self1, w_neigh1, b1.reshape(1, h), w_self2, w_neigh2,
    b2.reshape(1, c))


def _sc_layer2_epilogue(n, c, np_):
  nw = _NC * _NS
  main = (n // (nw * 8)) * 8
  extra = n - nw * main
  assert extra % 8 == 0 and extra >= 0
  buf = main + extra

  mesh = plsc.VectorSubcoreMesh(core_axis_name="c", subcore_axis_name="s")

  @functools.partial(
      pl.kernel, mesh=mesh,
      out_type=jax.ShapeDtypeStruct((n, c), jnp.float32),
      scratch_types=[
          pltpu.VMEM((buf, c), jnp.float32),
          pltpu.VMEM((buf, 16), jnp.float32),
          pltpu.VMEM((buf, c), jnp.float32),
          pltpu.VMEM((buf, c), jnp.float32),
          pltpu.VMEM((buf, c), jnp.float32),
      ],
      compiler_params=pltpu.CompilerParams(use_tc_tiling_on_sc=False))
  def body(hs2b, rdeg, agg2, h2, hb_v, rd_v, a0_v, a1_v, out_v):
    cid = lax.axis_index("c")
    sid = lax.axis_index("s")
    wid = cid * _NS + sid
    t0 = wid * main
    last = wid == nw - 1

    pltpu.sync_copy(hs2b.at[pl.ds(t0, main)], hb_v.at[pl.ds(0, main)])
    pltpu.sync_copy(rdeg.at[pl.ds(t0, main)], rd_v.at[pl.ds(0, main)])
    pltpu.sync_copy(agg2.at[0, pl.ds(t0, main)], a0_v.at[pl.ds(0, main)])
    pltpu.sync_copy(agg2.at[1, pl.ds(t0, main)], a1_v.at[pl.ds(0, main)])
    if extra:
      t1 = nw * main

      @pl.when(last)
      def _():
        pltpu.sync_copy(hs2b.at[pl.ds(t1, extra)], hb_v.at[pl.ds(main, extra)])
        pltpu.sync_copy(rdeg.at[pl.ds(t1, extra)], rd_v.at[pl.ds(main, extra)])
        pltpu.sync_copy(agg2.at[0, pl.ds(t1, extra)],
                        a0_v.at[pl.ds(main, extra)])
        pltpu.sync_copy(agg2.at[1, pl.ds(t1, extra)],
                        a1_v.at[pl.ds(main, extra)])

    nrows = jnp.where(last, main + extra, main)

    def row(r, carry):
      out_v[r] = hb_v[r] + (a0_v[r] + a1_v[r]) * rd_v[r]
      return carry

    lax.fori_loop(0, nrows, row, 0)

    pltpu.sync_copy(out_v.at[pl.ds(0, main)], h2.at[pl.ds(t0, main)])
    if extra:
      t1 = nw * main

      @pl.when(last)
      def _():
        pltpu.sync_copy(out_v.at[pl.ds(main, extra)], h2.at[pl.ds(t1, extra)])

  return body


def _tc_layer2(hs2, agg2, deg, b2):
  n, c = hs2.shape

  def tcc(hs2_b, agg2_b, deg_b, b2_b, h2_b):
    degs = jnp.maximum(deg_b[0, :, 0] + deg_b[1, :, 0], 1.0)
    h2_b[...] = hs2_b[...] + (agg2_b[0] + agg2_b[1]) / degs[:, None] + b2_b[...]

  return pl.pallas_call(
      tcc,
      grid=(1,),
      in_specs=[
          pl.BlockSpec((n, c), lambda i: (0, 0)),
          pl.BlockSpec((_NC, n, c), lambda i: (0, 0, 0)),
          pl.BlockSpec((_NC, n, _DW), lambda i: (0, 0, 0)),
          pl.BlockSpec((1, c), lambda i: (0, 0)),
      ],
      out_specs=pl.BlockSpec((n, c), lambda i: (0, 0)),
      out_shape=jax.ShapeDtypeStruct((n, c), jnp.float32),
  )(hs2, agg2, deg, b2.reshape(1, c))


def kernel(x, edge_index, W_self1, W_neigh1, b1, W_self2, W_neigh2, b2):
  n, d = x.shape
  e = edge_index.shape[1]
  c = W_self2.shape[0]
  rows_pt = _pad_rows(n) // _NS

  nw = _NC * _NS
  chunks = e // (nw * _K)
  esrc1 = edge_index[0].reshape(nw, chunks, _K)
  esrc2 = edge_index[0].reshape(nw, chunks // _G2, _G2 * _K)
  edst = edge_index[1].reshape(nw, chunks, _K)
  zacc = jnp.zeros((rows_pt, d), jnp.float32)
  zdeg = jnp.zeros((rows_pt, _DW), jnp.float32)
  zacc2 = jnp.zeros((rows_pt, c), jnp.float32)
  ones = jnp.ones((_K, _DW), jnp.float32)

  agg1, deg = _make_sc_agg(n, e, d, True)(x, esrc1, edst, zacc, zdeg, ones)
  h1, h1r, p2, hs2b, rdeg = _tc_layer1(x, agg1, deg, W_self1, W_neigh1, b1,
                                       W_self2, W_neigh2, b2)
  agg2 = _make_sc_agg(n, e, c, False, group=_G2)(
      p2, esrc2, edst, zacc2, zdeg, ones)[0]
  h2 = _sc_layer2_epilogue(n, c, _pad_rows(n))(hs2b, rdeg, agg2)
  return (h2, h1, h1r)
